# baseline (speedup 1.0000x reference)
import functools

import jax
import jax.numpy as jnp
from jax import lax
from jax.experimental import pallas as pl
from jax.experimental.pallas import tpu as pltpu
from jax.experimental.pallas import tpu_sc as plsc

N = 10000
E = 320000
NPAD = 10240
C = 512
EPAD = 327680
NROWS = EPAD // C
ZB = 128
NSC = 2
NSUB = 16
RPN = NPAD // NSUB
SL = 64
NQ = 4

_SELU_ALPHA = 1.6732632423543772
_SELU_SCALE = 1.0507009873554805


def _selu(x):
    return _SELU_SCALE * jnp.where(x > 0, x, _SELU_ALPHA * (jnp.exp(x) - 1.0))


def _zero_fill(ref, rows, width):
    @pl.loop(0, rows)
    def _(r):
        for cb in range(width // 16):
            ref[r, pl.ds(cb * 16, 16)] = jnp.zeros((16,), jnp.float32)


def _sc_mesh():
    return plsc.VectorSubcoreMesh(core_axis_name="c", subcore_axis_name="s")


_SC_PARAMS = pltpu.CompilerParams(use_tc_tiling_on_sc=False)


def _sc_phase0(ea, dst2d):
    rpw = NROWS // (NSC * NSUB)

    @functools.partial(
        pl.kernel,
        out_type=(
            jax.ShapeDtypeStruct((NSC, NPAD, 16), jnp.float32),
            jax.ShapeDtypeStruct((NSC, NPAD, 16), jnp.float32),
        ),
        mesh=_sc_mesh(),
        compiler_params=_SC_PARAMS,
        scratch_types=[
            pltpu.VMEM((rpw, C), jnp.int32),
            pltpu.VMEM((C, 16), jnp.float32),
            pltpu.VMEM((C, 16), jnp.float32),
            pltpu.VMEM_SHARED((NPAD, 16), jnp.float32),
            pltpu.VMEM_SHARED((NPAD, 16), jnp.float32),
        ],
    )
    def k(ea_hbm, dst_hbm, eag_out, deg_out, dstbuf, eabuf, onesbuf,
          acc_ea, acc_dg):
        c = lax.axis_index("c")
        s = lax.axis_index("s")
        row0 = (c * NSUB + s) * rpw

        pltpu.sync_copy(dst_hbm.at[pl.ds(row0, rpw)], dstbuf)

        _zero_fill(eabuf, C, 16)
        @pl.loop(0, C)
        def _(r):
            onesbuf[r, pl.ds(0, 16)] = jnp.ones((16,), jnp.float32)
        for k8 in range(RPN // ZB):
            pltpu.sync_copy(eabuf.at[pl.ds(0, ZB)],
                            acc_ea.at[pl.ds(s * RPN + k8 * ZB, ZB)])
            pltpu.sync_copy(eabuf.at[pl.ds(0, ZB)],
                            acc_dg.at[pl.ds(s * RPN + k8 * ZB, ZB)])
        plsc.subcore_barrier()

        @pl.loop(0, rpw)
        def _(j):
            pltpu.sync_copy(ea_hbm.at[pl.ds((row0 + j) * C, C)], eabuf)
            pltpu.sync_copy(eabuf, acc_ea.at[dstbuf.at[j]], add=True)
            pltpu.sync_copy(onesbuf, acc_dg.at[dstbuf.at[j]], add=True)

        plsc.subcore_barrier()
        pltpu.sync_copy(acc_ea.at[pl.ds(s * RPN, RPN)],
                        eag_out.at[c, pl.ds(s * RPN, RPN)])
        pltpu.sync_copy(acc_dg.at[pl.ds(s * RPN, RPN)],
                        deg_out.at[c, pl.ds(s * RPN, RPN)])

    return k(ea, dst2d)


def _sc_seg_slice(table, srcq2d, dst2d, dep):
    rpw = NROWS // (NSC * NSUB)

    @functools.partial(
        pl.kernel,
        out_type=jax.ShapeDtypeStruct((NSC, NPAD, SL), jnp.float32),
        mesh=_sc_mesh(),
        compiler_params=_SC_PARAMS,
        scratch_types=[
            pltpu.VMEM((rpw, C), jnp.int32),
            pltpu.VMEM((rpw, C), jnp.int32),
            pltpu.VMEM((C, SL), jnp.float32),
            pltpu.VMEM((C, SL), jnp.float32),
            pltpu.VMEM_SHARED((NPAD, SL), jnp.float32),
            pltpu.SemaphoreType.DMA,
            pltpu.SemaphoreType.DMA,
            pltpu.SemaphoreType.DMA,
            pltpu.SemaphoreType.DMA,
        ],
    )
    def k(h_hbm, src_hbm, dst_hbm, dep_hbm, s_out, srcbuf, dstbuf,
          g0, g1, acc, sg0, sg1, ss0, ss1):
        del dep_hbm
        c = lax.axis_index("c")
        s = lax.axis_index("s")
        row0 = (c * NSUB + s) * rpw
        NB = 2
        bufs = (g0, g1)
        gsems = (sg0, sg1)
        ssems = (ss0, ss1)

        pltpu.sync_copy(src_hbm.at[pl.ds(row0, rpw)], srcbuf)
        pltpu.sync_copy(dst_hbm.at[pl.ds(row0, rpw)], dstbuf)

        _zero_fill(g0, ZB, SL)
        for k8 in range(RPN // ZB):
            pltpu.sync_copy(g0.at[pl.ds(0, ZB)],
                            acc.at[pl.ds(s * RPN + k8 * ZB, ZB)])
        plsc.subcore_barrier()

        for b in range(NB):
            pltpu.async_copy(h_hbm.at[srcbuf.at[b]], bufs[b], gsems[b])

        @pl.loop(0, rpw, step=NB)
        def _(j):
            for b in range(NB):
                pltpu.make_async_copy(
                    h_hbm.at[srcbuf.at[j + b]], bufs[b], gsems[b]).wait()
                pltpu.async_copy(
                    bufs[b], acc.at[dstbuf.at[j + b]], ssems[b], add=True)
            for b in range(NB):
                pltpu.make_async_copy(
                    bufs[b], acc.at[dstbuf.at[j + b]], ssems[b]).wait()

                @pl.when(j + NB + b < rpw)
                def _():
                    pltpu.async_copy(
                        h_hbm.at[srcbuf.at[j + NB + b]], bufs[b], gsems[b])

        plsc.subcore_barrier()
        pltpu.sync_copy(acc.at[pl.ds(s * RPN, RPN)],
                        s_out.at[c, pl.ds(s * RPN, RPN)])

    return k(table, srcq2d, dst2d, dep)


def _seg_sum(hq, srcq2d_list, dst2d, nq, dep):
    flat = hq.reshape(NQ * NPAD, SL)
    outs = []
    for q in range(nq):
        out = _sc_seg_slice(flat, srcq2d_list[q], dst2d, dep)
        dep = lax.slice(out, (0, 0, 0), (2, NPAD, 16))
        outs.append(out)
    return outs


def _tc_layer(hq, s_list, eag_p, deg_p, ws, wn, we, b,
              *, din, act, final=False, wcls=None, bcls=None,
              s_projected=False):
    nqin = (din + SL - 1) // SL
    dout = ws.shape[1]
    nqout = (dout + SL - 1) // SL
    RB = 1280
    grid = (NPAD // RB,)
    ns = len(s_list)

    def body(h_ref, *rest):
        s_refs = rest[:ns]
        ea_ref, dg_ref, ws_ref, wn_ref, we_ref, b_ref = rest[ns:ns + 6]
        rest = rest[ns + 6:]
        if final:
            wcls_ref, bcls_ref, out_ref = rest
        else:
            (out_ref,) = rest

        f32 = jnp.float32
        hs = None
        for q in range(nqin):
            w = min(SL, din - q * SL)
            hq_blk = h_ref[q] if w == SL else h_ref[q][:, 0:w]
            term = jnp.dot(hq_blk, ws_ref[pl.ds(q * SL, w), :],
                           preferred_element_type=f32)
            hs = term if hs is None else hs + term

        if s_projected:
            sn = s_refs[0][0] + s_refs[0][1]
        else:
            sn = None
            for q in range(ns):
                w = min(SL, din - q * SL)
                sq = s_refs[q][0] + s_refs[q][1]
                if w < SL:
                    sq = sq[:, 0:w]
                term = jnp.dot(sq, wn_ref[pl.ds(q * SL, w), :],
                               preferred_element_type=f32)
                sn = term if sn is None else sn + term

        ea = ea_ref[0] + ea_ref[1]
        en = jnp.dot(ea, we_ref[...], preferred_element_type=f32)
        deg = dg_ref[0][:, 0:1] + dg_ref[1][:, 0:1]
        dinv = 1.0 / jnp.maximum(deg, 1.0)

        r = hs + (sn + en) * dinv + b_ref[...]
        if act:
            r = _selu(r)
        if final:
            feat = _selu(r)
            logits = jnp.sum(feat * wcls_ref[...], axis=1, keepdims=True)
            out_ref[...] = logits + bcls_ref[...]
        else:
            if dout < NQ * SL:
                r = jnp.concatenate(
                    [r, jnp.zeros((RB, NQ * SL - dout), f32)], axis=1)
            for q in range(NQ):
                out_ref[q] = r[:, q * SL:(q + 1) * SL]

    in_specs = [pl.BlockSpec((NQ, RB, SL), lambda i: (0, i, 0))]
    in_specs += [pl.BlockSpec((2, RB, SL), lambda i: (0, i, 0))] * ns
    in_specs += [
        pl.BlockSpec((2, RB, 16), lambda i: (0, i, 0)),
        pl.BlockSpec((2, RB, 16), lambda i: (0, i, 0)),
        pl.BlockSpec((din, dout), lambda i: (0, 0)),
        pl.BlockSpec((din, dout), lambda i: (0, 0)),
        pl.BlockSpec((16, dout), lambda i: (0, 0)),
        pl.BlockSpec((1, dout), lambda i: (0, 0)),
    ]
    args = [hq] + list(s_list) + [eag_p, deg_p, ws, wn, we,
                                  b.reshape(1, dout)]
    if final:
        in_specs.append(pl.BlockSpec((1, 256), lambda i: (0, 0)))
        in_specs.append(pl.BlockSpec((1, 1), lambda i: (0, 0)))
        args.append(wcls.reshape(1, 256))
        args.append(bcls.reshape(1, 1))
        out_shape = jax.ShapeDtypeStruct((NPAD, 1), jnp.float32)
        out_specs = pl.BlockSpec((RB, 1), lambda i: (i, 0))
    else:
        out_shape = jax.ShapeDtypeStruct((NQ, NPAD, SL), jnp.float32)
        out_specs = pl.BlockSpec((NQ, RB, SL), lambda i: (0, i, 0))

    return pl.pallas_call(
        body,
        grid=grid,
        in_specs=in_specs,
        out_specs=out_specs,
        out_shape=out_shape,
    )(*args)


def _tc_project(xpad, wn):
    RB = 1280
    dout = wn.shape[1]

    def body(x_ref, wn_ref, out_ref):
        p = jnp.dot(x_ref[...], wn_ref[...], preferred_element_type=jnp.float32)
        out_ref[0] = p
        for q in range(1, NQ):
            out_ref[q] = jnp.zeros((RB, SL), jnp.float32)

    return pl.pallas_call(
        body,
        grid=(NPAD // RB,),
        in_specs=[pl.BlockSpec((RB, xpad.shape[1]), lambda i: (i, 0)),
                  pl.BlockSpec((xpad.shape[1], dout), lambda i: (0, 0))],
        out_specs=pl.BlockSpec((NQ, RB, SL), lambda i: (0, i, 0)),
        out_shape=jax.ShapeDtypeStruct((NQ, NPAD, SL), jnp.float32),
    )(xpad, wn)


def kernel(x, edge_index, edge_attr,
           W_self_0, W_nbr_0, W_edge_0, b_0,
           W_self_1, W_nbr_1, W_edge_1, b_1,
           W_self_2, W_nbr_2, W_edge_2, b_2,
           W_self_3, W_nbr_3, W_edge_3, b_3,
           W_self_4, W_nbr_4, W_edge_4, b_4,
           W_cls, b_cls):
    src = jnp.pad(edge_index[0], (0, EPAD - E))
    dst2d = jnp.pad(edge_index[1], (0, EPAD - E),
                    constant_values=N).reshape(NROWS, C)
    srcq = [(src + q * NPAD).reshape(NROWS, C) for q in range(NQ)]
    ea_pad = jnp.pad(edge_attr, ((0, EPAD - E), (0, 0)))

    xpad = jnp.pad(x, ((0, NPAD - N), (0, 0)))
    xq = jnp.stack(
        [xpad[:, 0:SL], xpad[:, SL:2 * SL]]
        + [jnp.zeros((NPAD, SL), jnp.float32)] * (NQ - 2))

    eag_p, deg_p = _sc_phase0(ea_pad, dst2d)

    p0 = _tc_project(xpad, W_nbr_0)
    s0 = _seg_sum(p0, srcq, dst2d, 1, eag_p)
    h1 = _tc_layer(xq, s0, eag_p, deg_p, W_self_0, W_nbr_0, W_edge_0, b_0,
                   din=128, act=True, s_projected=True)
    s1 = _seg_sum(h1, srcq, dst2d, 1,
                  lax.slice(s0[-1], (0, 0, 0), (2, NPAD, 16)))
    h2 = _tc_layer(h1, s1, eag_p, deg_p, W_self_1, W_nbr_1, W_edge_1, b_1,
                   din=64, act=True)
    s2 = _seg_sum(h2, srcq, dst2d, 2,
                  lax.slice(s1[-1], (0, 0, 0), (2, NPAD, 16)))
    h3 = _tc_layer(h2, s2, eag_p, deg_p, W_self_2, W_nbr_2, W_edge_2, b_2,
                   din=128, act=True)
    s3 = _seg_sum(h3, srcq, dst2d, 4,
                  lax.slice(s2[-1], (0, 0, 0), (2, NPAD, 16)))
    h4 = _tc_layer(h3, s3, eag_p, deg_p, W_self_3, W_nbr_3, W_edge_3, b_3,
                   din=256, act=True)
    s4 = _seg_sum(h4, srcq, dst2d, 4,
                  lax.slice(s3[-1], (0, 0, 0), (2, NPAD, 16)))
    out = _tc_layer(h4, s4, eag_p, deg_p, W_self_4, W_nbr_4, W_edge_4, b_4,
                    din=256, act=False, final=True, wcls=W_cls, bcls=b_cls)
    return out[:N]


# speedup vs baseline: 1.0886x; 1.0886x over previous
import functools

import jax
import jax.numpy as jnp
from jax import lax
from jax.experimental import pallas as pl
from jax.experimental.pallas import tpu as pltpu
from jax.experimental.pallas import tpu_sc as plsc

N = 10000
E = 320000
NPAD = 10240
C = 512
EPAD = 327680
NROWS = EPAD // C
ZB = 128
NSC = 2
NSUB = 16
RPN = NPAD // NSUB
SL = 64
NQ = 4

_SELU_ALPHA = 1.6732632423543772
_SELU_SCALE = 1.0507009873554805


def _selu(x):
    return _SELU_SCALE * jnp.where(x > 0, x, _SELU_ALPHA * (jnp.exp(x) - 1.0))


def _zero_fill(ref, rows, width):
    @pl.loop(0, rows)
    def _(r):
        for cb in range(width // 16):
            ref[r, pl.ds(cb * 16, 16)] = jnp.zeros((16,), jnp.float32)


def _sc_mesh():
    return plsc.VectorSubcoreMesh(core_axis_name="c", subcore_axis_name="s")


_SC_PARAMS = pltpu.CompilerParams(use_tc_tiling_on_sc=False)


def _sc_phase0(ea, dst2d):
    rpw = NROWS // (NSC * NSUB)

    @functools.partial(
        pl.kernel,
        out_type=(
            jax.ShapeDtypeStruct((NSC, NPAD, 16), jnp.float32),
            jax.ShapeDtypeStruct((NSC, NPAD, 16), jnp.float32),
        ),
        mesh=_sc_mesh(),
        compiler_params=_SC_PARAMS,
        scratch_types=[
            pltpu.VMEM((rpw, C), jnp.int32),
            pltpu.VMEM((C, 16), jnp.float32),
            pltpu.VMEM((C, 16), jnp.float32),
            pltpu.VMEM_SHARED((NPAD, 16), jnp.float32),
            pltpu.VMEM_SHARED((NPAD, 16), jnp.float32),
        ],
    )
    def k(ea_hbm, dst_hbm, eag_out, deg_out, dstbuf, eabuf, onesbuf,
          acc_ea, acc_dg):
        c = lax.axis_index("c")
        s = lax.axis_index("s")
        row0 = (c * NSUB + s) * rpw

        pltpu.sync_copy(dst_hbm.at[pl.ds(row0, rpw)], dstbuf)

        _zero_fill(eabuf, C, 16)
        @pl.loop(0, C)
        def _(r):
            onesbuf[r, pl.ds(0, 16)] = jnp.ones((16,), jnp.float32)
        for k8 in range(RPN // ZB):
            pltpu.sync_copy(eabuf.at[pl.ds(0, ZB)],
                            acc_ea.at[pl.ds(s * RPN + k8 * ZB, ZB)])
            pltpu.sync_copy(eabuf.at[pl.ds(0, ZB)],
                            acc_dg.at[pl.ds(s * RPN + k8 * ZB, ZB)])
        plsc.subcore_barrier()

        @pl.loop(0, rpw)
        def _(j):
            pltpu.sync_copy(ea_hbm.at[pl.ds((row0 + j) * C, C)], eabuf)
            pltpu.sync_copy(eabuf, acc_ea.at[dstbuf.at[j]], add=True)
            pltpu.sync_copy(onesbuf, acc_dg.at[dstbuf.at[j]], add=True)

        plsc.subcore_barrier()
        pltpu.sync_copy(acc_ea.at[pl.ds(s * RPN, RPN)],
                        eag_out.at[c, pl.ds(s * RPN, RPN)])
        pltpu.sync_copy(acc_dg.at[pl.ds(s * RPN, RPN)],
                        deg_out.at[c, pl.ds(s * RPN, RPN)])

    return k(ea, dst2d)


def _sc_seg_slice(table, srcq2d, dst2d):
    rpw = NROWS // (NSC * NSUB)

    @functools.partial(
        pl.kernel,
        out_type=jax.ShapeDtypeStruct((NSC, NPAD, SL), jnp.float32),
        mesh=_sc_mesh(),
        compiler_params=_SC_PARAMS,
        scratch_types=[
            pltpu.VMEM((rpw, C), jnp.int32),
            pltpu.VMEM((rpw, C), jnp.int32),
            pltpu.VMEM((C, SL), jnp.float32),
            pltpu.VMEM((C, SL), jnp.float32),
            pltpu.VMEM_SHARED((NPAD, SL), jnp.float32),
            pltpu.SemaphoreType.DMA,
            pltpu.SemaphoreType.DMA,
            pltpu.SemaphoreType.DMA,
            pltpu.SemaphoreType.DMA,
        ],
    )
    def k(h_hbm, src_hbm, dst_hbm, s_out, srcbuf, dstbuf,
          g0, g1, acc, sg0, sg1, ss0, ss1):
        c = lax.axis_index("c")
        s = lax.axis_index("s")
        row0 = (c * NSUB + s) * rpw
        NB = 2
        bufs = (g0, g1)
        gsems = (sg0, sg1)
        ssems = (ss0, ss1)

        pltpu.sync_copy(src_hbm.at[pl.ds(row0, rpw)], srcbuf)
        pltpu.sync_copy(dst_hbm.at[pl.ds(row0, rpw)], dstbuf)

        _zero_fill(g0, ZB, SL)
        for k8 in range(RPN // ZB):
            pltpu.sync_copy(g0.at[pl.ds(0, ZB)],
                            acc.at[pl.ds(s * RPN + k8 * ZB, ZB)])
        plsc.subcore_barrier()

        for b in range(NB):
            pltpu.async_copy(h_hbm.at[srcbuf.at[b]], bufs[b], gsems[b])

        @pl.loop(0, rpw, step=NB)
        def _(j):
            for b in range(NB):
                pltpu.make_async_copy(
                    h_hbm.at[srcbuf.at[j + b]], bufs[b], gsems[b]).wait()
                pltpu.async_copy(
                    bufs[b], acc.at[dstbuf.at[j + b]], ssems[b], add=True)
            for b in range(NB):
                pltpu.make_async_copy(
                    bufs[b], acc.at[dstbuf.at[j + b]], ssems[b]).wait()

                @pl.when(j + NB + b < rpw)
                def _():
                    pltpu.async_copy(
                        h_hbm.at[srcbuf.at[j + NB + b]], bufs[b], gsems[b])

        plsc.subcore_barrier()
        pltpu.sync_copy(acc.at[pl.ds(s * RPN, RPN)],
                        s_out.at[c, pl.ds(s * RPN, RPN)])

    return k(table, srcq2d, dst2d)


def _seg_sum(hq, srcq2d_list, dst2d, nq):
    flat = hq.reshape(NQ * NPAD, SL)
    return [_sc_seg_slice(flat, srcq2d_list[q], dst2d) for q in range(nq)]


def _tc_layer(hq, s_list, eag_p, deg_p, ws, wn, we, b,
              *, din, act, final=False, wcls=None, bcls=None,
              s_projected=False):
    nqin = (din + SL - 1) // SL
    dout = ws.shape[1]
    nqout = (dout + SL - 1) // SL
    RB = 1280
    grid = (NPAD // RB,)
    ns = len(s_list)

    def body(h_ref, *rest):
        s_refs = rest[:ns]
        ea_ref, dg_ref, ws_ref, wn_ref, we_ref, b_ref = rest[ns:ns + 6]
        rest = rest[ns + 6:]
        if final:
            wcls_ref, bcls_ref, out_ref = rest
        else:
            (out_ref,) = rest

        f32 = jnp.float32
        hs = None
        for q in range(nqin):
            w = min(SL, din - q * SL)
            hq_blk = h_ref[q] if w == SL else h_ref[q][:, 0:w]
            term = jnp.dot(hq_blk, ws_ref[pl.ds(q * SL, w), :],
                           preferred_element_type=f32)
            hs = term if hs is None else hs + term

        if s_projected:
            sn = s_refs[0][0] + s_refs[0][1]
        else:
            sn = None
            for q in range(ns):
                w = min(SL, din - q * SL)
                sq = s_refs[q][0] + s_refs[q][1]
                if w < SL:
                    sq = sq[:, 0:w]
                term = jnp.dot(sq, wn_ref[pl.ds(q * SL, w), :],
                               preferred_element_type=f32)
                sn = term if sn is None else sn + term

        ea = ea_ref[0] + ea_ref[1]
        en = jnp.dot(ea, we_ref[...], preferred_element_type=f32)
        deg = dg_ref[0][:, 0:1] + dg_ref[1][:, 0:1]
        dinv = 1.0 / jnp.maximum(deg, 1.0)

        r = hs + (sn + en) * dinv + b_ref[...]
        if act:
            r = _selu(r)
        if final:
            feat = _selu(r)
            logits = jnp.sum(feat * wcls_ref[...], axis=1, keepdims=True)
            out_ref[...] = logits + bcls_ref[...]
        else:
            if dout < NQ * SL:
                r = jnp.concatenate(
                    [r, jnp.zeros((RB, NQ * SL - dout), f32)], axis=1)
            for q in range(NQ):
                out_ref[q] = r[:, q * SL:(q + 1) * SL]

    in_specs = [pl.BlockSpec((NQ, RB, SL), lambda i: (0, i, 0))]
    in_specs += [pl.BlockSpec((2, RB, SL), lambda i: (0, i, 0))] * ns
    in_specs += [
        pl.BlockSpec((2, RB, 16), lambda i: (0, i, 0)),
        pl.BlockSpec((2, RB, 16), lambda i: (0, i, 0)),
        pl.BlockSpec((din, dout), lambda i: (0, 0)),
        pl.BlockSpec((din, dout), lambda i: (0, 0)),
        pl.BlockSpec((16, dout), lambda i: (0, 0)),
        pl.BlockSpec((1, dout), lambda i: (0, 0)),
    ]
    args = [hq] + list(s_list) + [eag_p, deg_p, ws, wn, we,
                                  b.reshape(1, dout)]
    if final:
        in_specs.append(pl.BlockSpec((1, 256), lambda i: (0, 0)))
        in_specs.append(pl.BlockSpec((1, 1), lambda i: (0, 0)))
        args.append(wcls.reshape(1, 256))
        args.append(bcls.reshape(1, 1))
        out_shape = jax.ShapeDtypeStruct((NPAD, 1), jnp.float32)
        out_specs = pl.BlockSpec((RB, 1), lambda i: (i, 0))
    else:
        out_shape = jax.ShapeDtypeStruct((NQ, NPAD, SL), jnp.float32)
        out_specs = pl.BlockSpec((NQ, RB, SL), lambda i: (0, i, 0))

    return pl.pallas_call(
        body,
        grid=grid,
        in_specs=in_specs,
        out_specs=out_specs,
        out_shape=out_shape,
    )(*args)


def _tc_project(xpad, wn):
    RB = 1280
    dout = wn.shape[1]

    def body(x_ref, wn_ref, out_ref):
        p = jnp.dot(x_ref[...], wn_ref[...], preferred_element_type=jnp.float32)
        out_ref[0] = p
        for q in range(1, NQ):
            out_ref[q] = jnp.zeros((RB, SL), jnp.float32)

    return pl.pallas_call(
        body,
        grid=(NPAD // RB,),
        in_specs=[pl.BlockSpec((RB, xpad.shape[1]), lambda i: (i, 0)),
                  pl.BlockSpec((xpad.shape[1], dout), lambda i: (0, 0))],
        out_specs=pl.BlockSpec((NQ, RB, SL), lambda i: (0, i, 0)),
        out_shape=jax.ShapeDtypeStruct((NQ, NPAD, SL), jnp.float32),
    )(xpad, wn)


def kernel(x, edge_index, edge_attr,
           W_self_0, W_nbr_0, W_edge_0, b_0,
           W_self_1, W_nbr_1, W_edge_1, b_1,
           W_self_2, W_nbr_2, W_edge_2, b_2,
           W_self_3, W_nbr_3, W_edge_3, b_3,
           W_self_4, W_nbr_4, W_edge_4, b_4,
           W_cls, b_cls):
    src = jnp.pad(edge_index[0], (0, EPAD - E))
    dst2d = jnp.pad(edge_index[1], (0, EPAD - E),
                    constant_values=N).reshape(NROWS, C)
    srcq = [(src + q * NPAD).reshape(NROWS, C) for q in range(NQ)]
    ea_pad = jnp.pad(edge_attr, ((0, EPAD - E), (0, 0)))

    xpad = jnp.pad(x, ((0, NPAD - N), (0, 0)))
    xq = jnp.stack(
        [xpad[:, 0:SL], xpad[:, SL:2 * SL]]
        + [jnp.zeros((NPAD, SL), jnp.float32)] * (NQ - 2))

    eag_p, deg_p = _sc_phase0(ea_pad, dst2d)

    p0 = _tc_project(xpad, W_nbr_0)
    s0 = _seg_sum(p0, srcq, dst2d, 1)
    h1 = _tc_layer(xq, s0, eag_p, deg_p, W_self_0, W_nbr_0, W_edge_0, b_0,
                   din=128, act=True, s_projected=True)
    s1 = _seg_sum(h1, srcq, dst2d, 1)
    h2 = _tc_layer(h1, s1, eag_p, deg_p, W_self_1, W_nbr_1, W_edge_1, b_1,
                   din=64, act=True)
    s2 = _seg_sum(h2, srcq, dst2d, 2)
    h3 = _tc_layer(h2, s2, eag_p, deg_p, W_self_2, W_nbr_2, W_edge_2, b_2,
                   din=128, act=True)
    s3 = _seg_sum(h3, srcq, dst2d, 4)
    h4 = _tc_layer(h3, s3, eag_p, deg_p, W_self_3, W_nbr_3, W_edge_3, b_3,
                   din=256, act=True)
    s4 = _seg_sum(h4, srcq, dst2d, 4)
    out = _tc_layer(h4, s4, eag_p, deg_p, W_self_4, W_nbr_4, W_edge_4, b_4,
                    din=256, act=False, final=True, wcls=W_cls, bcls=b_cls)
    return out[:N]


# speedup vs baseline: 1.0928x; 1.0039x over previous
import functools

import jax
import jax.numpy as jnp
from jax import lax
from jax.experimental import pallas as pl
from jax.experimental.pallas import tpu as pltpu
from jax.experimental.pallas import tpu_sc as plsc

N = 10000
E = 320000
NPAD = 10240
C = 512
EPAD = 327680
NROWS = EPAD // C
ZB = 128
NSC = 2
NSUB = 16
RPN = NPAD // NSUB
SL = 64
NQ = 4

_SELU_ALPHA = 1.6732632423543772
_SELU_SCALE = 1.0507009873554805


def _selu(x):
    return _SELU_SCALE * jnp.where(x > 0, x, _SELU_ALPHA * (jnp.exp(x) - 1.0))


def _zero_fill(ref, rows, width):
    @pl.loop(0, rows)
    def _(r):
        for cb in range(width // 16):
            ref[r, pl.ds(cb * 16, 16)] = jnp.zeros((16,), jnp.float32)


def _sc_mesh():
    return plsc.VectorSubcoreMesh(core_axis_name="c", subcore_axis_name="s")


_SC_PARAMS = pltpu.CompilerParams(use_tc_tiling_on_sc=False)


def _sc_phase0(ea, dst2d):
    rpw = NROWS // (NSC * NSUB)

    @functools.partial(
        pl.kernel,
        out_type=(
            jax.ShapeDtypeStruct((NSC, NPAD, 16), jnp.float32),
            jax.ShapeDtypeStruct((NSC, NPAD, 16), jnp.float32),
        ),
        mesh=_sc_mesh(),
        compiler_params=_SC_PARAMS,
        scratch_types=[
            pltpu.VMEM((rpw, C), jnp.int32),
            pltpu.VMEM((C, 16), jnp.float32),
            pltpu.VMEM((C, 16), jnp.float32),
            pltpu.VMEM((C, 16), jnp.float32),
            pltpu.VMEM_SHARED((NPAD, 16), jnp.float32),
            pltpu.VMEM_SHARED((NPAD, 16), jnp.float32),
            pltpu.SemaphoreType.DMA,
            pltpu.SemaphoreType.DMA,
            pltpu.SemaphoreType.DMA,
            pltpu.SemaphoreType.DMA,
            pltpu.SemaphoreType.DMA,
            pltpu.SemaphoreType.DMA,
        ],
    )
    def k(ea_hbm, dst_hbm, eag_out, deg_out, dstbuf, ea0, ea1, onesbuf,
          acc_ea, acc_dg, l0, l1, se0, se1, so0, so1):
        c = lax.axis_index("c")
        s = lax.axis_index("s")
        row0 = (c * NSUB + s) * rpw
        eab = (ea0, ea1)
        lsem = (l0, l1)
        esem = (se0, se1)
        osem = (so0, so1)

        pltpu.sync_copy(dst_hbm.at[pl.ds(row0, rpw)], dstbuf)

        _zero_fill(ea0, C, 16)
        @pl.loop(0, C)
        def _(r):
            onesbuf[r, pl.ds(0, 16)] = jnp.ones((16,), jnp.float32)
        for k8 in range(RPN // ZB):
            pltpu.sync_copy(ea0.at[pl.ds(0, ZB)],
                            acc_ea.at[pl.ds(s * RPN + k8 * ZB, ZB)])
            pltpu.sync_copy(ea0.at[pl.ds(0, ZB)],
                            acc_dg.at[pl.ds(s * RPN + k8 * ZB, ZB)])
        plsc.subcore_barrier()

        for b in range(2):
            pltpu.async_copy(ea_hbm.at[pl.ds((row0 + b) * C, C)],
                             eab[b], lsem[b])

        @pl.loop(0, rpw, step=2)
        def _(j):
            for b in range(2):
                pltpu.make_async_copy(
                    ea_hbm.at[pl.ds((row0 + j + b) * C, C)],
                    eab[b], lsem[b]).wait()
                pltpu.async_copy(eab[b], acc_ea.at[dstbuf.at[j + b]],
                                 esem[b], add=True)
                pltpu.async_copy(onesbuf, acc_dg.at[dstbuf.at[j + b]],
                                 osem[b], add=True)
            for b in range(2):
                pltpu.make_async_copy(
                    eab[b], acc_ea.at[dstbuf.at[j + b]], esem[b]).wait()
                pltpu.make_async_copy(
                    onesbuf, acc_dg.at[dstbuf.at[j + b]], osem[b]).wait()

                @pl.when(j + 2 + b < rpw)
                def _():
                    pltpu.async_copy(
                        ea_hbm.at[pl.ds((row0 + j + 2 + b) * C, C)],
                        eab[b], lsem[b])

        plsc.subcore_barrier()
        pltpu.sync_copy(acc_ea.at[pl.ds(s * RPN, RPN)],
                        eag_out.at[c, pl.ds(s * RPN, RPN)])
        pltpu.sync_copy(acc_dg.at[pl.ds(s * RPN, RPN)],
                        deg_out.at[c, pl.ds(s * RPN, RPN)])

    return k(ea, dst2d)


def _sc_seg_slice(table, srcq2d, dst2d):
    rpw = NROWS // (NSC * NSUB)

    @functools.partial(
        pl.kernel,
        out_type=jax.ShapeDtypeStruct((NSC, NPAD, SL), jnp.float32),
        mesh=_sc_mesh(),
        compiler_params=_SC_PARAMS,
        scratch_types=[
            pltpu.VMEM((rpw, C), jnp.int32),
            pltpu.VMEM((rpw, C), jnp.int32),
            pltpu.VMEM((C, SL), jnp.float32),
            pltpu.VMEM((C, SL), jnp.float32),
            pltpu.VMEM_SHARED((NPAD, SL), jnp.float32),
            pltpu.SemaphoreType.DMA,
            pltpu.SemaphoreType.DMA,
            pltpu.SemaphoreType.DMA,
            pltpu.SemaphoreType.DMA,
        ],
    )
    def k(h_hbm, src_hbm, dst_hbm, s_out, srcbuf, dstbuf,
          g0, g1, acc, sg0, sg1, ss0, ss1):
        c = lax.axis_index("c")
        s = lax.axis_index("s")
        row0 = (c * NSUB + s) * rpw
        NB = 2
        bufs = (g0, g1)
        gsems = (sg0, sg1)
        ssems = (ss0, ss1)

        pltpu.sync_copy(src_hbm.at[pl.ds(row0, rpw)], srcbuf)
        pltpu.sync_copy(dst_hbm.at[pl.ds(row0, rpw)], dstbuf)

        _zero_fill(g0, ZB, SL)
        for k8 in range(RPN // ZB):
            pltpu.sync_copy(g0.at[pl.ds(0, ZB)],
                            acc.at[pl.ds(s * RPN + k8 * ZB, ZB)])
        plsc.subcore_barrier()

        for b in range(NB):
            pltpu.async_copy(h_hbm.at[srcbuf.at[b]], bufs[b], gsems[b])

        @pl.loop(0, rpw, step=NB)
        def _(j):
            for b in range(NB):
                pltpu.make_async_copy(
                    h_hbm.at[srcbuf.at[j + b]], bufs[b], gsems[b]).wait()
                pltpu.async_copy(
                    bufs[b], acc.at[dstbuf.at[j + b]], ssems[b], add=True)
            for b in range(NB):
                pltpu.make_async_copy(
                    bufs[b], acc.at[dstbuf.at[j + b]], ssems[b]).wait()

                @pl.when(j + NB + b < rpw)
                def _():
                    pltpu.async_copy(
                        h_hbm.at[srcbuf.at[j + NB + b]], bufs[b], gsems[b])

        plsc.subcore_barrier()
        pltpu.sync_copy(acc.at[pl.ds(s * RPN, RPN)],
                        s_out.at[c, pl.ds(s * RPN, RPN)])

    return k(table, srcq2d, dst2d)


def _seg_sum(hq, srcq2d_list, dst2d, nq):
    flat = hq.reshape(NQ * NPAD, SL)
    return [_sc_seg_slice(flat, srcq2d_list[q], dst2d) for q in range(nq)]


def _tc_layer(hq, s_list, eag_p, deg_p, ws, wn, we, b,
              *, din, act, final=False, wcls=None, bcls=None,
              s_projected=False):
    nqin = (din + SL - 1) // SL
    dout = ws.shape[1]
    nqout = (dout + SL - 1) // SL
    RB = 1280
    grid = (NPAD // RB,)
    ns = len(s_list)

    def body(h_ref, *rest):
        s_refs = rest[:ns]
        ea_ref, dg_ref, ws_ref, wn_ref, we_ref, b_ref = rest[ns:ns + 6]
        rest = rest[ns + 6:]
        if final:
            wcls_ref, bcls_ref, out_ref = rest
        else:
            (out_ref,) = rest

        f32 = jnp.float32
        hs = None
        for q in range(nqin):
            w = min(SL, din - q * SL)
            hq_blk = h_ref[q] if w == SL else h_ref[q][:, 0:w]
            term = jnp.dot(hq_blk, ws_ref[pl.ds(q * SL, w), :],
                           preferred_element_type=f32)
            hs = term if hs is None else hs + term

        if s_projected:
            sn = s_refs[0][0] + s_refs[0][1]
        else:
            sn = None
            for q in range(ns):
                w = min(SL, din - q * SL)
                sq = s_refs[q][0] + s_refs[q][1]
                if w < SL:
                    sq = sq[:, 0:w]
                term = jnp.dot(sq, wn_ref[pl.ds(q * SL, w), :],
                               preferred_element_type=f32)
                sn = term if sn is None else sn + term

        ea = ea_ref[0] + ea_ref[1]
        en = jnp.dot(ea, we_ref[...], preferred_element_type=f32)
        deg = dg_ref[0][:, 0:1] + dg_ref[1][:, 0:1]
        dinv = 1.0 / jnp.maximum(deg, 1.0)

        r = hs + (sn + en) * dinv + b_ref[...]
        if act:
            r = _selu(r)
        if final:
            feat = _selu(r)
            logits = jnp.sum(feat * wcls_ref[...], axis=1, keepdims=True)
            out_ref[...] = logits + bcls_ref[...]
        else:
            if dout < NQ * SL:
                r = jnp.concatenate(
                    [r, jnp.zeros((RB, NQ * SL - dout), f32)], axis=1)
            for q in range(NQ):
                out_ref[q] = r[:, q * SL:(q + 1) * SL]

    in_specs = [pl.BlockSpec((NQ, RB, SL), lambda i: (0, i, 0))]
    in_specs += [pl.BlockSpec((2, RB, SL), lambda i: (0, i, 0))] * ns
    in_specs += [
        pl.BlockSpec((2, RB, 16), lambda i: (0, i, 0)),
        pl.BlockSpec((2, RB, 16), lambda i: (0, i, 0)),
        pl.BlockSpec((din, dout), lambda i: (0, 0)),
        pl.BlockSpec((din, dout), lambda i: (0, 0)),
        pl.BlockSpec((16, dout), lambda i: (0, 0)),
        pl.BlockSpec((1, dout), lambda i: (0, 0)),
    ]
    args = [hq] + list(s_list) + [eag_p, deg_p, ws, wn, we,
                                  b.reshape(1, dout)]
    if final:
        in_specs.append(pl.BlockSpec((1, 256), lambda i: (0, 0)))
        in_specs.append(pl.BlockSpec((1, 1), lambda i: (0, 0)))
        args.append(wcls.reshape(1, 256))
        args.append(bcls.reshape(1, 1))
        out_shape = jax.ShapeDtypeStruct((NPAD, 1), jnp.float32)
        out_specs = pl.BlockSpec((RB, 1), lambda i: (i, 0))
    else:
        out_shape = jax.ShapeDtypeStruct((NQ, NPAD, SL), jnp.float32)
        out_specs = pl.BlockSpec((NQ, RB, SL), lambda i: (0, i, 0))

    return pl.pallas_call(
        body,
        grid=grid,
        in_specs=in_specs,
        out_specs=out_specs,
        out_shape=out_shape,
    )(*args)


def _tc_project(xpad, wn):
    RB = 1280
    dout = wn.shape[1]

    def body(x_ref, wn_ref, out_ref):
        p = jnp.dot(x_ref[...], wn_ref[...], preferred_element_type=jnp.float32)
        out_ref[0] = p
        for q in range(1, NQ):
            out_ref[q] = jnp.zeros((RB, SL), jnp.float32)

    return pl.pallas_call(
        body,
        grid=(NPAD // RB,),
        in_specs=[pl.BlockSpec((RB, xpad.shape[1]), lambda i: (i, 0)),
                  pl.BlockSpec((xpad.shape[1], dout), lambda i: (0, 0))],
        out_specs=pl.BlockSpec((NQ, RB, SL), lambda i: (0, i, 0)),
        out_shape=jax.ShapeDtypeStruct((NQ, NPAD, SL), jnp.float32),
    )(xpad, wn)


def kernel(x, edge_index, edge_attr,
           W_self_0, W_nbr_0, W_edge_0, b_0,
           W_self_1, W_nbr_1, W_edge_1, b_1,
           W_self_2, W_nbr_2, W_edge_2, b_2,
           W_self_3, W_nbr_3, W_edge_3, b_3,
           W_self_4, W_nbr_4, W_edge_4, b_4,
           W_cls, b_cls):
    src = jnp.pad(edge_index[0], (0, EPAD - E))
    dst2d = jnp.pad(edge_index[1], (0, EPAD - E),
                    constant_values=N).reshape(NROWS, C)
    srcq = [(src + q * NPAD).reshape(NROWS, C) for q in range(NQ)]
    ea_pad = jnp.pad(edge_attr, ((0, EPAD - E), (0, 0)))

    xpad = jnp.pad(x, ((0, NPAD - N), (0, 0)))
    xq = jnp.stack(
        [xpad[:, 0:SL], xpad[:, SL:2 * SL]]
        + [jnp.zeros((NPAD, SL), jnp.float32)] * (NQ - 2))

    eag_p, deg_p = _sc_phase0(ea_pad, dst2d)

    p0 = _tc_project(xpad, W_nbr_0)
    s0 = _seg_sum(p0, srcq, dst2d, 1)
    h1 = _tc_layer(xq, s0, eag_p, deg_p, W_self_0, W_nbr_0, W_edge_0, b_0,
                   din=128, act=True, s_projected=True)
    s1 = _seg_sum(h1, srcq, dst2d, 1)
    h2 = _tc_layer(h1, s1, eag_p, deg_p, W_self_1, W_nbr_1, W_edge_1, b_1,
                   din=64, act=True)
    s2 = _seg_sum(h2, srcq, dst2d, 2)
    h3 = _tc_layer(h2, s2, eag_p, deg_p, W_self_2, W_nbr_2, W_edge_2, b_2,
                   din=128, act=True)
    s3 = _seg_sum(h3, srcq, dst2d, 4)
    h4 = _tc_layer(h3, s3, eag_p, deg_p, W_self_3, W_nbr_3, W_edge_3, b_3,
                   din=256, act=True)
    s4 = _seg_sum(h4, srcq, dst2d, 4)
    out = _tc_layer(h4, s4, eag_p, deg_p, W_self_4, W_nbr_4, W_edge_4, b_4,
                    din=256, act=False, final=True, wcls=W_cls, bcls=b_cls)
    return out[:N]
